# Initial kernel scaffold; baseline (speedup 1.0000x reference)
import jax
import jax.numpy as jnp
from jax.experimental import pallas as pl


def kernel(p1, x1, W1, gamma1, beta1, W2, gamma2, beta2):
    raise NotImplementedError("write your pallas kernel here")


# speedup vs baseline: 7.4108x; 7.4108x over previous
import functools

import jax
import jax.numpy as jnp
from jax import lax
from jax.experimental import pallas as pl
from jax.experimental.pallas import tpu as pltpu
from jax.experimental.pallas import tpu_sc as plsc

_B, _N, _COORD = 4, 2048, 3
_CIN, _COUT = 128, 256
_M = _N // 4
_K = 16
_EPS = 1e-5
_P = _B * _M * _K
_NPOS = float(_P)

_NT = 512
_MT = 128
_PT = 512
_MROWS = _PT // _K
_GRID_MLP = _P // _PT

_F32 = jnp.float32
_I32 = jnp.int32


def _fps_body(px_ref, py_ref, pz_ref, ox_ref, oy_ref, oz_ref):
    px = px_ref[...]
    py = py_ref[...]
    pz = pz_ref[...]
    io_n = lax.broadcasted_iota(_I32, (_B, _N), 1)
    io_m = lax.broadcasted_iota(_I32, (_B, _M), 1)

    def step(i, carry):
        dist, far, ax, ay, az = carry
        sel = io_n == far
        cx = jnp.sum(jnp.where(sel, px, 0.0), axis=1, keepdims=True)
        cy = jnp.sum(jnp.where(sel, py, 0.0), axis=1, keepdims=True)
        cz = jnp.sum(jnp.where(sel, pz, 0.0), axis=1, keepdims=True)
        rec = io_m == i
        ax = jnp.where(rec, cx, ax)
        ay = jnp.where(rec, cy, ay)
        az = jnp.where(rec, cz, az)
        dx = px - cx
        dy = py - cy
        dz = pz - cz
        d = dx * dx + dy * dy + dz * dz
        dist = jnp.minimum(dist, d)
        mx = jnp.max(dist, axis=1, keepdims=True)
        far = jnp.min(jnp.where(dist == mx, io_n, _N), axis=1, keepdims=True)
        return dist, far, ax, ay, az

    init = (
        jnp.full((_B, _N), 1e10, _F32),
        jnp.zeros((_B, 1), _I32),
        jnp.zeros((_B, _M), _F32),
        jnp.zeros((_B, _M), _F32),
        jnp.zeros((_B, _M), _F32),
    )
    _, _, ax, ay, az = lax.fori_loop(0, _M, step, init)
    ox_ref[...] = ax
    oy_ref[...] = ay
    oz_ref[...] = az


def _run_fps(px, py, pz):
    out = jax.ShapeDtypeStruct((_B, _M), _F32)
    return pl.pallas_call(
        _fps_body,
        out_shape=[out, out, out],
    )(px, py, pz)


def _table_body(p1_ref, x1t_ref, w1pt_ref, w1xt_ref, o_ref):
    x1b = x1t_ref[...].reshape(_NT, _CIN)
    mm = jnp.dot(x1b, w1xt_ref[...], preferred_element_type=_F32)
    p1b = p1_ref[...].reshape(_NT, _COORD)
    w = w1pt_ref[...]
    pc = (p1b[:, 0:1] * w[0:1, :]
          + p1b[:, 1:2] * w[1:2, :]
          + p1b[:, 2:3] * w[2:3, :])
    o_ref[...] = mm + pc


def _run_table(p1, x1t, w1pt, w1xt):
    nblk = _N // _NT
    return pl.pallas_call(
        _table_body,
        grid=(_B, nblk),
        in_specs=[
            pl.BlockSpec((1, _NT, _COORD), lambda b, j: (b, j, 0)),
            pl.BlockSpec((1, _NT, _CIN), lambda b, j: (b, j, 0)),
            pl.BlockSpec((_COORD, _COUT), lambda b, j: (0, 0)),
            pl.BlockSpec((_CIN, _COUT), lambda b, j: (0, 0)),
        ],
        out_specs=pl.BlockSpec((_NT, _COUT), lambda b, j: (b * nblk + j, 0)),
        out_shape=jax.ShapeDtypeStruct((_B * _N, _COUT), _F32),
    )(p1, x1t, w1pt, w1xt)


def _knn_body(px_ref, py_ref, pz_ref, qx_ref, qy_ref, qz_ref, o_ref):
    b = pl.program_id(0)
    px = px_ref[...].reshape(1, _N)
    py = py_ref[...].reshape(1, _N)
    pz = pz_ref[...].reshape(1, _N)
    qx = qx_ref[...].reshape(_MT, 1)
    qy = qy_ref[...].reshape(_MT, 1)
    qz = qz_ref[...].reshape(_MT, 1)
    x2 = px * px + py * py + pz * pz
    s = qx * px + qy * py + qz * pz
    d = x2 - (s + s)
    zm = (px == 0.0) & (py == 0.0) & (pz == 0.0)
    d = jnp.where(zm, jnp.inf, d)
    io = lax.broadcasted_iota(_I32, (_MT, _N), 1)
    io_k = lax.broadcasted_iota(_I32, (_MT, _K), 1)
    acc = jnp.zeros((_MT, _K), _I32)
    for k in range(_K):
        mn = jnp.min(d, axis=1, keepdims=True)
        fi = jnp.min(jnp.where(d == mn, io, _N), axis=1, keepdims=True)
        acc = jnp.where(io_k == k, fi, acc)
        d = jnp.where(io == fi, jnp.inf, d)
    o_ref[...] = acc + b * _N


def _run_knn(px, py, pz, q3x, q3y, q3z):
    nblk = _M // _MT
    qspec = pl.BlockSpec((1, _MT, 1), lambda b, j: (b, j, 0))
    pspec = pl.BlockSpec((1, 1, _N), lambda b, j: (b, 0, 0))
    return pl.pallas_call(
        _knn_body,
        grid=(_B, nblk),
        in_specs=[pspec, pspec, pspec, qspec, qspec, qspec],
        out_specs=pl.BlockSpec((_MT, _K), lambda b, j: (b * nblk + j, 0)),
        out_shape=jax.ShapeDtypeStruct((_B * _M, _K), _I32),
    )(px[:, None, :], py[:, None, :], pz[:, None, :], q3x, q3y, q3z)


_SC_NC = 2
_SC_NS = 16
_SC_NW = _SC_NC * _SC_NS
_SC_ROWS = _P // _SC_NW
_SC_CH = 128
_SC_NCHUNK = _SC_ROWS // _SC_CH


def _sc_gather_body(table_hbm, idx_hbm, out_hbm, idx_v, rows_v, sem):
    wid = lax.axis_index("s") * _SC_NC + lax.axis_index("c")
    base = wid * _SC_ROWS

    def chunk(c, carry):
        off = base + c * _SC_CH
        pltpu.sync_copy(idx_hbm.at[pl.ds(off, _SC_CH)], idx_v)
        pltpu.async_copy(table_hbm.at[idx_v], rows_v, sem).wait()
        pltpu.sync_copy(rows_v, out_hbm.at[pl.ds(off, _SC_CH)])
        return carry

    lax.fori_loop(0, _SC_NCHUNK, chunk, 0)


def _sc_gather(table, idx_flat):
    mesh = plsc.VectorSubcoreMesh(core_axis_name="c", subcore_axis_name="s")
    f = functools.partial(
        pl.kernel,
        mesh=mesh,
        out_type=jax.ShapeDtypeStruct((_P, _COUT), _F32),
        scratch_types=[
            pltpu.VMEM((_SC_CH,), _I32),
            pltpu.VMEM((_SC_CH, _COUT), _F32),
            pltpu.SemaphoreType.DMA,
        ],
    )(_sc_gather_body)
    return f(table, idx_flat)


def _query_term(p2b, w):
    return (p2b[:, 0:1] * w[0:1, :]
            + p2b[:, 1:2] * w[1:2, :]
            + p2b[:, 2:3] * w[2:3, :])


def _stats1_body(g_ref, p2_ref, w1pt_ref, s_ref, q_ref):
    g = g_ref[...]
    c = _query_term(p2_ref[...], w1pt_ref[...])
    h = (g.reshape(_MROWS, _K, _COUT) - c[:, None, :]).reshape(_PT, _COUT)
    s_ref[...] = jnp.sum(h, axis=0, keepdims=True).reshape(1, 1, _COUT)
    q_ref[...] = jnp.sum(h * h, axis=0, keepdims=True).reshape(1, 1, _COUT)


def _bn_scale_shift(s_ref, q_ref, gamma_ref, beta_ref):
    stot = jnp.sum(s_ref[...], axis=0)
    qtot = jnp.sum(q_ref[...], axis=0)
    mean = stot / _NPOS
    var = qtot / _NPOS - mean * mean
    scale = gamma_ref[...] * lax.rsqrt(var + _EPS)
    shift = beta_ref[...] - mean * scale
    return scale, shift


def _mlp1_body(g_ref, p2_ref, w1pt_ref, s1_ref, q1_ref, g1_ref, b1_ref,
               w2t_ref, h2_ref, s2_ref, q2_ref):
    scale, shift = _bn_scale_shift(s1_ref, q1_ref, g1_ref, b1_ref)
    g = g_ref[...]
    c = _query_term(p2_ref[...], w1pt_ref[...])
    h = (g.reshape(_MROWS, _K, _COUT) - c[:, None, :]).reshape(_PT, _COUT)
    a = jnp.maximum(h * scale + shift, 0.0)
    h2 = jnp.dot(a, w2t_ref[...], preferred_element_type=_F32)
    h2_ref[...] = h2
    s2_ref[...] = jnp.sum(h2, axis=0, keepdims=True).reshape(1, 1, _COUT)
    q2_ref[...] = jnp.sum(h2 * h2, axis=0, keepdims=True).reshape(1, 1, _COUT)


def _mlp2_body(h2_ref, s2_ref, q2_ref, g2_ref, b2_ref, o_ref):
    scale, shift = _bn_scale_shift(s2_ref, q2_ref, g2_ref, b2_ref)
    y = jnp.maximum(h2_ref[...] * scale + shift, 0.0)
    o_ref[...] = jnp.max(y.reshape(_MROWS, _K, _COUT), axis=1)


def _run_mlp(G, p2f, w1pt, gamma1, beta1, w2t, gamma2, beta2):
    gspec = pl.BlockSpec((_PT, _COUT), lambda t: (t, 0))
    p2spec = pl.BlockSpec((_MROWS, _COORD), lambda t: (t, 0))
    wp_spec = pl.BlockSpec((_COORD, _COUT), lambda t: (0, 0))
    vec_spec = pl.BlockSpec((1, _COUT), lambda t: (0, 0))
    part_w = pl.BlockSpec((1, 1, _COUT), lambda t: (t, 0, 0))
    part_r = pl.BlockSpec((_GRID_MLP, 1, _COUT), lambda t: (0, 0, 0))
    part_shape = jax.ShapeDtypeStruct((_GRID_MLP, 1, _COUT), _F32)

    s1, q1 = pl.pallas_call(
        _stats1_body,
        grid=(_GRID_MLP,),
        in_specs=[gspec, p2spec, wp_spec],
        out_specs=[part_w, part_w],
        out_shape=[part_shape, part_shape],
    )(G, p2f, w1pt)

    h2, s2, q2 = pl.pallas_call(
        _mlp1_body,
        grid=(_GRID_MLP,),
        in_specs=[gspec, p2spec, wp_spec, part_r, part_r, vec_spec, vec_spec,
                  pl.BlockSpec((_COUT, _COUT), lambda t: (0, 0))],
        out_specs=[gspec, part_w, part_w],
        out_shape=[jax.ShapeDtypeStruct((_P, _COUT), _F32),
                   part_shape, part_shape],
    )(G, p2f, w1pt, s1, q1, gamma1, beta1, w2t)

    out = pl.pallas_call(
        _mlp2_body,
        grid=(_GRID_MLP,),
        in_specs=[gspec, part_r, part_r, vec_spec, vec_spec],
        out_specs=pl.BlockSpec((_MROWS, _COUT), lambda t: (t, 0)),
        out_shape=jax.ShapeDtypeStruct((_B * _M, _COUT), _F32),
    )(h2, s2, q2, gamma2, beta2)
    return out


def kernel(p1, x1, W1, gamma1, beta1, W2, gamma2, beta2):
    px = p1[:, :, 0]
    py = p1[:, :, 1]
    pz = p1[:, :, 2]

    p2x, p2y, p2z = _run_fps(px, py, pz)
    p2 = jnp.stack([p2x, p2y, p2z], axis=-1)

    w1pt = jnp.transpose(W1[:, :_COORD])
    w1xt = jnp.transpose(W1[:, _COORD:])
    x1t = jnp.transpose(x1, (0, 2, 1))
    table = _run_table(p1, x1t, w1pt, w1xt)

    knn = _run_knn(px, py, pz,
                   p2x[:, :, None], p2y[:, :, None], p2z[:, :, None])
    idx_flat = knn.reshape(_P)

    G = _sc_gather(table, idx_flat)

    p2f = p2.reshape(_B * _M, _COORD)
    out = _run_mlp(G, p2f, w1pt,
                   gamma1[None, :], beta1[None, :],
                   jnp.transpose(W2),
                   gamma2[None, :], beta2[None, :])

    new_features = jnp.transpose(out.reshape(_B, _M, _COUT), (0, 2, 1))
    return p2, new_features


# speedup vs baseline: 7.9535x; 1.0732x over previous
import functools

import jax
import jax.numpy as jnp
from jax import lax
from jax.experimental import pallas as pl
from jax.experimental.pallas import tpu as pltpu
from jax.experimental.pallas import tpu_sc as plsc

_B, _N, _COORD = 4, 2048, 3
_CIN, _COUT = 128, 256
_M = _N // 4
_K = 16
_EPS = 1e-5
_P = _B * _M * _K
_NPOS = float(_P)

_NT = 512
_MT = 128
_PT = 512
_MROWS = _PT // _K
_GRID_MLP = _P // _PT

_F32 = jnp.float32
_I32 = jnp.int32


def _fps_body(px_ref, py_ref, pz_ref, ox_ref, oy_ref, oz_ref):
    px = px_ref[...]
    py = py_ref[...]
    pz = pz_ref[...]
    io_n = lax.broadcasted_iota(_I32, (_B, _N), 1)
    io_f = lax.broadcasted_iota(_I32, (_B, 128), 1)
    zf = jnp.zeros((_B, 128), _F32)

    def step(i, carry):
        dist, far, acx, acy, acz = carry
        sel = io_n == far
        cx = jnp.sum(jnp.where(sel, px, 0.0), axis=1, keepdims=True)
        cy = jnp.sum(jnp.where(sel, py, 0.0), axis=1, keepdims=True)
        cz = jnp.sum(jnp.where(sel, pz, 0.0), axis=1, keepdims=True)
        rec = io_f == i % 128
        acx = jnp.where(rec, cx, acx)
        acy = jnp.where(rec, cy, acy)
        acz = jnp.where(rec, cz, acz)

        @pl.when(i % 128 == 127)
        def _flush():
            off = pl.multiple_of((i // 128) * 128, 128)
            ox_ref[:, pl.ds(off, 128)] = acx
            oy_ref[:, pl.ds(off, 128)] = acy
            oz_ref[:, pl.ds(off, 128)] = acz

        dx = px - cx
        dy = py - cy
        dz = pz - cz
        d = dx * dx + dy * dy + dz * dz
        dist = jnp.minimum(dist, d)
        mx = jnp.max(dist, axis=1, keepdims=True)
        far = jnp.min(jnp.where(dist == mx, io_n, _N), axis=1, keepdims=True)
        return dist, far, acx, acy, acz

    init = (
        jnp.full((_B, _N), 1e10, _F32),
        jnp.zeros((_B, 1), _I32),
        zf, zf, zf,
    )
    lax.fori_loop(0, _M, step, init)


def _run_fps(px, py, pz):
    out = jax.ShapeDtypeStruct((_B, _M), _F32)
    return pl.pallas_call(
        _fps_body,
        out_shape=[out, out, out],
    )(px, py, pz)


def _tabknn_body(p1_ref, x1_ref, w1pt_ref, w1xt_ref,
                 px_ref, py_ref, pz_ref, qx_ref, qy_ref, qz_ref,
                 t_ref, o_ref):
    b = pl.program_id(0)
    x1b = x1_ref[...].reshape(_CIN, _NT)
    mm = lax.dot_general(x1b, w1xt_ref[...], (((0,), (0,)), ((), ())),
                         preferred_element_type=_F32)
    p1b = p1_ref[...].reshape(_NT, _COORD)
    w = w1pt_ref[...]
    pc = (p1b[:, 0:1] * w[0:1, :]
          + p1b[:, 1:2] * w[1:2, :]
          + p1b[:, 2:3] * w[2:3, :])
    t_ref[...] = mm + pc
    px = px_ref[...].reshape(1, _N)
    py = py_ref[...].reshape(1, _N)
    pz = pz_ref[...].reshape(1, _N)
    qx = qx_ref[...].reshape(_MT, 1)
    qy = qy_ref[...].reshape(_MT, 1)
    qz = qz_ref[...].reshape(_MT, 1)
    x2 = px * px + py * py + pz * pz
    s = qx * px + qy * py + qz * pz
    d = x2 - (s + s)
    zm = (px == 0.0) & (py == 0.0) & (pz == 0.0)
    d = jnp.where(zm, jnp.inf, d)
    io = lax.broadcasted_iota(_I32, (_MT, _N), 1)
    io_k = lax.broadcasted_iota(_I32, (_MT, _K), 1)
    acc = jnp.zeros((_MT, _K), _I32)
    for k in range(_K):
        mn = jnp.min(d, axis=1, keepdims=True)
        fi = jnp.min(jnp.where(d == mn, io, _N), axis=1, keepdims=True)
        acc = jnp.where(io_k == k, fi, acc)
        d = jnp.where(io == fi, jnp.inf, d)
    o_ref[...] = acc + b * _N


def _run_table_knn(p1, x1, w1pt, w1xt, px, py, pz, q3x, q3y, q3z):
    nblk = _N // _NT
    qspec = pl.BlockSpec((1, _MT, 1), lambda b, j: (b, j, 0))
    pspec = pl.BlockSpec((1, 1, _N), lambda b, j: (b, 0, 0))
    return pl.pallas_call(
        _tabknn_body,
        grid=(_B, nblk),
        in_specs=[
            pl.BlockSpec((1, _NT, _COORD), lambda b, j: (b, j, 0)),
            pl.BlockSpec((1, _CIN, _NT), lambda b, j: (b, 0, j)),
            pl.BlockSpec((_COORD, _COUT), lambda b, j: (0, 0)),
            pl.BlockSpec((_CIN, _COUT), lambda b, j: (0, 0)),
            pspec, pspec, pspec, qspec, qspec, qspec,
        ],
        out_specs=[
            pl.BlockSpec((_NT, _COUT), lambda b, j: (b * nblk + j, 0)),
            pl.BlockSpec((_MT, _K), lambda b, j: (b * nblk + j, 0)),
        ],
        out_shape=[
            jax.ShapeDtypeStruct((_B * _N, _COUT), _F32),
            jax.ShapeDtypeStruct((_B * _M, _K), _I32),
        ],
    )(p1, x1, w1pt, w1xt,
      px[:, None, :], py[:, None, :], pz[:, None, :], q3x, q3y, q3z)


_SC_NC = 2
_SC_NS = 16
_SC_NW = _SC_NC * _SC_NS
_SC_ROWS = _P // _SC_NW
_SC_CH = 128
_SC_NCHUNK = _SC_ROWS // _SC_CH


def _sc_gather_body(table_hbm, idx_hbm, out_hbm, idx_v, rows_v, sem):
    wid = lax.axis_index("s") * _SC_NC + lax.axis_index("c")
    base = wid * _SC_ROWS

    def chunk(c, carry):
        off = base + c * _SC_CH
        pltpu.sync_copy(idx_hbm.at[pl.ds(off, _SC_CH)], idx_v)
        pltpu.async_copy(table_hbm.at[idx_v], rows_v, sem).wait()
        pltpu.sync_copy(rows_v, out_hbm.at[pl.ds(off, _SC_CH)])
        return carry

    lax.fori_loop(0, _SC_NCHUNK, chunk, 0)


def _sc_gather(table, idx_flat):
    mesh = plsc.VectorSubcoreMesh(core_axis_name="c", subcore_axis_name="s")
    f = functools.partial(
        pl.kernel,
        mesh=mesh,
        out_type=jax.ShapeDtypeStruct((_P, _COUT), _F32),
        scratch_types=[
            pltpu.VMEM((_SC_CH,), _I32),
            pltpu.VMEM((_SC_CH, _COUT), _F32),
            pltpu.SemaphoreType.DMA,
        ],
    )(_sc_gather_body)
    return f(table, idx_flat)


def _query_term(p2b, w):
    return (p2b[:, 0:1] * w[0:1, :]
            + p2b[:, 1:2] * w[1:2, :]
            + p2b[:, 2:3] * w[2:3, :])


def _stats1_body(g_ref, p2_ref, w1pt_ref, s_ref, q_ref):
    g = g_ref[...]
    c = _query_term(p2_ref[...], w1pt_ref[...])
    h = (g.reshape(_MROWS, _K, _COUT) - c[:, None, :]).reshape(_PT, _COUT)
    s_ref[...] = jnp.sum(h, axis=0, keepdims=True).reshape(1, 1, _COUT)
    q_ref[...] = jnp.sum(h * h, axis=0, keepdims=True).reshape(1, 1, _COUT)


def _bn_scale_shift(s_ref, q_ref, gamma_ref, beta_ref):
    stot = jnp.sum(s_ref[...], axis=0)
    qtot = jnp.sum(q_ref[...], axis=0)
    mean = stot / _NPOS
    var = qtot / _NPOS - mean * mean
    scale = gamma_ref[...] * lax.rsqrt(var + _EPS)
    shift = beta_ref[...] - mean * scale
    return scale, shift


def _mlp1_body(g_ref, p2_ref, w1pt_ref, s1_ref, q1_ref, g1_ref, b1_ref,
               w2t_ref, h2_ref, s2_ref, q2_ref):
    scale, shift = _bn_scale_shift(s1_ref, q1_ref, g1_ref, b1_ref)
    g = g_ref[...]
    c = _query_term(p2_ref[...], w1pt_ref[...])
    h = (g.reshape(_MROWS, _K, _COUT) - c[:, None, :]).reshape(_PT, _COUT)
    a = jnp.maximum(h * scale + shift, 0.0)
    h2 = jnp.dot(a, w2t_ref[...], preferred_element_type=_F32)
    h2_ref[...] = h2
    s2_ref[...] = jnp.sum(h2, axis=0, keepdims=True).reshape(1, 1, _COUT)
    q2_ref[...] = jnp.sum(h2 * h2, axis=0, keepdims=True).reshape(1, 1, _COUT)


_MROWS2 = 128
_PT2 = _MROWS2 * _K
_GRID_MLP2 = _P // _PT2


def _mlp2_body(h2_ref, s2_ref, q2_ref, g2_ref, b2_ref, o_ref):
    scale, shift = _bn_scale_shift(s2_ref, q2_ref, g2_ref, b2_ref)
    y = jnp.maximum(h2_ref[...] * scale + shift, 0.0)
    pooled = jnp.max(y.reshape(_MROWS2, _K, _COUT), axis=1)
    o_ref[...] = jnp.transpose(pooled)[None]


def _run_mlp(G, p2f, w1pt, gamma1, beta1, w2t, gamma2, beta2):
    gspec = pl.BlockSpec((_PT, _COUT), lambda t: (t, 0))
    p2spec = pl.BlockSpec((_MROWS, _COORD), lambda t: (t, 0))
    wp_spec = pl.BlockSpec((_COORD, _COUT), lambda t: (0, 0))
    vec_spec = pl.BlockSpec((1, _COUT), lambda t: (0, 0))
    part_w = pl.BlockSpec((1, 1, _COUT), lambda t: (t, 0, 0))
    part_r = pl.BlockSpec((_GRID_MLP, 1, _COUT), lambda t: (0, 0, 0))
    part_shape = jax.ShapeDtypeStruct((_GRID_MLP, 1, _COUT), _F32)

    s1, q1 = pl.pallas_call(
        _stats1_body,
        grid=(_GRID_MLP,),
        in_specs=[gspec, p2spec, wp_spec],
        out_specs=[part_w, part_w],
        out_shape=[part_shape, part_shape],
    )(G, p2f, w1pt)

    h2, s2, q2 = pl.pallas_call(
        _mlp1_body,
        grid=(_GRID_MLP,),
        in_specs=[gspec, p2spec, wp_spec, part_r, part_r, vec_spec, vec_spec,
                  pl.BlockSpec((_COUT, _COUT), lambda t: (0, 0))],
        out_specs=[gspec, part_w, part_w],
        out_shape=[jax.ShapeDtypeStruct((_P, _COUT), _F32),
                   part_shape, part_shape],
    )(G, p2f, w1pt, s1, q1, gamma1, beta1, w2t)

    mchunks = _M // _MROWS2
    out = pl.pallas_call(
        _mlp2_body,
        grid=(_GRID_MLP2,),
        in_specs=[pl.BlockSpec((_PT2, _COUT), lambda t: (t, 0)),
                  part_r, part_r, vec_spec, vec_spec],
        out_specs=pl.BlockSpec((1, _COUT, _MROWS2),
                               lambda t: (t // mchunks, 0, t % mchunks)),
        out_shape=jax.ShapeDtypeStruct((_B, _COUT, _M), _F32),
    )(h2, s2, q2, gamma2, beta2)
    return out


def kernel(p1, x1, W1, gamma1, beta1, W2, gamma2, beta2):
    px = p1[:, :, 0]
    py = p1[:, :, 1]
    pz = p1[:, :, 2]

    p2x, p2y, p2z = _run_fps(px, py, pz)
    p2 = jnp.stack([p2x, p2y, p2z], axis=-1)

    w1pt = jnp.transpose(W1[:, :_COORD])
    w1xt = jnp.transpose(W1[:, _COORD:])
    table, knn = _run_table_knn(
        p1, x1, w1pt, w1xt, px, py, pz,
        p2x[:, :, None], p2y[:, :, None], p2z[:, :, None])
    idx_flat = knn.reshape(_P)

    G = _sc_gather(table, idx_flat)

    p2f = p2.reshape(_B * _M, _COORD)
    new_features = _run_mlp(G, p2f, w1pt,
                            gamma1[None, :], beta1[None, :],
                            jnp.transpose(W2),
                            gamma2[None, :], beta2[None, :])
    return p2, new_features


# speedup vs baseline: 8.1794x; 1.0284x over previous
import functools

import jax
import jax.numpy as jnp
from jax import lax
from jax.experimental import pallas as pl
from jax.experimental.pallas import tpu as pltpu
from jax.experimental.pallas import tpu_sc as plsc

_B, _N, _COORD = 4, 2048, 3
_CIN, _COUT = 128, 256
_M = _N // 4
_K = 16
_EPS = 1e-5
_P = _B * _M * _K
_NPOS = float(_P)

_NT = 512
_MT = 128
_PT = 512
_MROWS = _PT // _K
_GRID_MLP = _P // _PT

_F32 = jnp.float32
_I32 = jnp.int32


def _fps_body(px_ref, py_ref, pz_ref, ox_ref, oy_ref, oz_ref):
    px = px_ref[...]
    py = py_ref[...]
    pz = pz_ref[...]
    io_n = lax.broadcasted_iota(_I32, (_B, _N), 1)
    io_f = lax.broadcasted_iota(_I32, (_B, 128), 1)
    zf = jnp.zeros((_B, 128), _F32)

    def step(i, carry):
        dist, far, acx, acy, acz = carry
        sel = io_n == far
        cx = jnp.sum(jnp.where(sel, px, 0.0), axis=1, keepdims=True)
        cy = jnp.sum(jnp.where(sel, py, 0.0), axis=1, keepdims=True)
        cz = jnp.sum(jnp.where(sel, pz, 0.0), axis=1, keepdims=True)
        rec = io_f == i % 128
        acx = jnp.where(rec, cx, acx)
        acy = jnp.where(rec, cy, acy)
        acz = jnp.where(rec, cz, acz)

        @pl.when(i % 128 == 127)
        def _flush():
            off = pl.multiple_of((i // 128) * 128, 128)
            ox_ref[:, pl.ds(off, 128)] = acx
            oy_ref[:, pl.ds(off, 128)] = acy
            oz_ref[:, pl.ds(off, 128)] = acz

        dx = px - cx
        dy = py - cy
        dz = pz - cz
        d = dx * dx + dy * dy + dz * dz
        dist = jnp.minimum(dist, d)
        mx = jnp.max(dist, axis=1, keepdims=True)
        far = jnp.min(jnp.where(dist == mx, io_n, _N), axis=1, keepdims=True)
        return dist, far, acx, acy, acz

    init = (
        jnp.full((_B, _N), 1e10, _F32),
        jnp.zeros((_B, 1), _I32),
        zf, zf, zf,
    )
    lax.fori_loop(0, _M, step, init)


def _run_fps(px, py, pz):
    out = jax.ShapeDtypeStruct((_B, _M), _F32)
    return pl.pallas_call(
        _fps_body,
        out_shape=[out, out, out],
    )(px, py, pz)


def _tabknn_body(p1_ref, x1_ref, w1pt_ref, w1xt_ref,
                 px_ref, py_ref, pz_ref, qx_ref, qy_ref, qz_ref,
                 t_ref, o_ref):
    b = pl.program_id(0)
    x1b = x1_ref[...].reshape(_CIN, _NT)
    mm = lax.dot_general(x1b, w1xt_ref[...], (((0,), (0,)), ((), ())),
                         preferred_element_type=_F32)
    p1b = p1_ref[...].reshape(_NT, _COORD)
    w = w1pt_ref[...]
    pc = (p1b[:, 0:1] * w[0:1, :]
          + p1b[:, 1:2] * w[1:2, :]
          + p1b[:, 2:3] * w[2:3, :])
    t_ref[...] = mm + pc
    px = px_ref[...].reshape(1, _N)
    py = py_ref[...].reshape(1, _N)
    pz = pz_ref[...].reshape(1, _N)
    qx = qx_ref[...].reshape(_MT, 1)
    qy = qy_ref[...].reshape(_MT, 1)
    qz = qz_ref[...].reshape(_MT, 1)
    x2 = px * px + py * py + pz * pz
    s = qx * px + qy * py + qz * pz
    d = x2 - (s + s)
    zm = (px == 0.0) & (py == 0.0) & (pz == 0.0)
    d = jnp.where(zm, jnp.inf, d)
    io = lax.broadcasted_iota(_I32, (_MT, _N), 1)
    io_k = lax.broadcasted_iota(_I32, (_MT, _K), 1)
    acc = jnp.zeros((_MT, _K), _I32)
    for k in range(_K):
        mn = jnp.min(d, axis=1, keepdims=True)
        fi = jnp.min(jnp.where(d == mn, io, _N), axis=1, keepdims=True)
        acc = jnp.where(io_k == k, fi, acc)
        d = jnp.where(io == fi, jnp.inf, d)
    o_ref[...] = acc + b * _N


def _run_table_knn(p1, x1, w1pt, w1xt, px, py, pz, q3x, q3y, q3z):
    nblk = _N // _NT
    qspec = pl.BlockSpec((1, _MT, 1), lambda b, j: (b, j, 0))
    pspec = pl.BlockSpec((1, 1, _N), lambda b, j: (b, 0, 0))
    return pl.pallas_call(
        _tabknn_body,
        grid=(_B, nblk),
        in_specs=[
            pl.BlockSpec((1, _NT, _COORD), lambda b, j: (b, j, 0)),
            pl.BlockSpec((1, _CIN, _NT), lambda b, j: (b, 0, j)),
            pl.BlockSpec((_COORD, _COUT), lambda b, j: (0, 0)),
            pl.BlockSpec((_CIN, _COUT), lambda b, j: (0, 0)),
            pspec, pspec, pspec, qspec, qspec, qspec,
        ],
        out_specs=[
            pl.BlockSpec((_NT, _COUT), lambda b, j: (b * nblk + j, 0)),
            pl.BlockSpec((_MT, _K), lambda b, j: (b * nblk + j, 0)),
        ],
        out_shape=[
            jax.ShapeDtypeStruct((_B * _N, _COUT), _F32),
            jax.ShapeDtypeStruct((_B * _M, _K), _I32),
        ],
    )(p1, x1, w1pt, w1xt,
      px[:, None, :], py[:, None, :], pz[:, None, :], q3x, q3y, q3z)


_SC_NC = 2
_SC_NS = 16
_SC_NW = _SC_NC * _SC_NS
_SC_ROWS = _P // _SC_NW
_SC_CH = 128
_SC_NCHUNK = _SC_ROWS // _SC_CH


def _sc_gather_body(table_hbm, idx_hbm, out_hbm, idx_v, rows_v, sem):
    wid = lax.axis_index("s") * _SC_NC + lax.axis_index("c")
    base = wid * _SC_ROWS

    def chunk(c, carry):
        off = base + c * _SC_CH
        pltpu.sync_copy(idx_hbm.at[pl.ds(off, _SC_CH)], idx_v)
        pltpu.async_copy(table_hbm.at[idx_v], rows_v, sem).wait()
        pltpu.sync_copy(rows_v, out_hbm.at[pl.ds(off, _SC_CH)])
        return carry

    lax.fori_loop(0, _SC_NCHUNK, chunk, 0)


def _sc_gather(table, idx_flat):
    mesh = plsc.VectorSubcoreMesh(core_axis_name="c", subcore_axis_name="s")
    f = functools.partial(
        pl.kernel,
        mesh=mesh,
        out_type=jax.ShapeDtypeStruct((_P, _COUT), _F32),
        scratch_types=[
            pltpu.VMEM((_SC_CH,), _I32),
            pltpu.VMEM((_SC_CH, _COUT), _F32),
            pltpu.SemaphoreType.DMA,
        ],
    )(_sc_gather_body)
    return f(table, idx_flat)


def _query_term(p2b, w):
    return (p2b[:, 0:1] * w[0:1, :]
            + p2b[:, 1:2] * w[1:2, :]
            + p2b[:, 2:3] * w[2:3, :])


_MROWS2 = 128
_PT2 = _MROWS2 * _K
_NOUT = _P // _PT2
_NT_MLP = _GRID_MLP


def _bn_scale_shift(acc_ref, gamma_ref, beta_ref):
    mean = acc_ref[0:1, :] / _NPOS
    var = acc_ref[1:2, :] / _NPOS - mean * mean
    scale = gamma_ref[...] * lax.rsqrt(var + _EPS)
    shift = beta_ref[...] - mean * scale
    return scale, shift


def _mlp_body(g_ref, p2_ref, w1pt_ref, g1_ref, b1_ref, w2t_ref,
              g2_ref, b2_ref, o_ref, h2_ref, acc1_ref, acc2_ref):
    p = pl.program_id(0)
    t = pl.program_id(1)

    @pl.when((p == 0) & (t == 0))
    def _init():
        acc1_ref[...] = jnp.zeros((2, _COUT), _F32)
        acc2_ref[...] = jnp.zeros((2, _COUT), _F32)

    def _h():
        g = g_ref[...]
        c = _query_term(p2_ref[...], w1pt_ref[...])
        return (g.reshape(_MROWS, _K, _COUT) - c[:, None, :]).reshape(
            _PT, _COUT)

    @pl.when(p == 0)
    def _phase_stats():
        h = _h()
        s = jnp.sum(h, axis=0, keepdims=True)
        q = jnp.sum(h * h, axis=0, keepdims=True)
        acc1_ref[...] += jnp.concatenate([s, q], axis=0)

    @pl.when(p == 1)
    def _phase_mlp1():
        scale, shift = _bn_scale_shift(acc1_ref, g1_ref, b1_ref)
        a = jnp.maximum(_h() * scale + shift, 0.0)
        h2 = jnp.dot(a, w2t_ref[...], preferred_element_type=_F32)
        h2_ref[pl.ds(t * _PT, _PT), :] = h2
        s = jnp.sum(h2, axis=0, keepdims=True)
        q = jnp.sum(h2 * h2, axis=0, keepdims=True)
        acc2_ref[...] += jnp.concatenate([s, q], axis=0)

    @pl.when((p == 2) & (t < _NOUT))
    def _phase_out():
        scale, shift = _bn_scale_shift(acc2_ref, g2_ref, b2_ref)
        h2 = h2_ref[pl.ds(t * _PT2, _PT2), :]
        y = jnp.maximum(h2 * scale + shift, 0.0)
        pooled = jnp.max(y.reshape(_MROWS2, _K, _COUT), axis=1)
        o_ref[...] = jnp.transpose(pooled)[None]


def _run_mlp(G, p2f, w1pt, gamma1, beta1, w2t, gamma2, beta2):
    pin = _NT_MLP - 1
    mchunks = _M // _MROWS2

    def gmap(p, t):
        return (jnp.where(p == 2, pin, t), 0)

    def omap(p, t):
        q = jnp.minimum(t, _NOUT - 1)
        return (q // mchunks, 0, q % mchunks)

    return pl.pallas_call(
        _mlp_body,
        grid=(3, _NT_MLP),
        in_specs=[
            pl.BlockSpec((_PT, _COUT), gmap),
            pl.BlockSpec((_MROWS, _COORD), gmap),
            pl.BlockSpec((_COORD, _COUT), lambda p, t: (0, 0)),
            pl.BlockSpec((1, _COUT), lambda p, t: (0, 0)),
            pl.BlockSpec((1, _COUT), lambda p, t: (0, 0)),
            pl.BlockSpec((_COUT, _COUT), lambda p, t: (0, 0)),
            pl.BlockSpec((1, _COUT), lambda p, t: (0, 0)),
            pl.BlockSpec((1, _COUT), lambda p, t: (0, 0)),
        ],
        out_specs=pl.BlockSpec((1, _COUT, _MROWS2), omap),
        out_shape=jax.ShapeDtypeStruct((_B, _COUT, _M), _F32),
        scratch_shapes=[
            pltpu.VMEM((_P, _COUT), _F32),
            pltpu.VMEM((2, _COUT), _F32),
            pltpu.VMEM((2, _COUT), _F32),
        ],
    )(G, p2f, w1pt, gamma1, beta1, w2t, gamma2, beta2)


def kernel(p1, x1, W1, gamma1, beta1, W2, gamma2, beta2):
    px = p1[:, :, 0]
    py = p1[:, :, 1]
    pz = p1[:, :, 2]

    p2x, p2y, p2z = _run_fps(px, py, pz)
    p2 = jnp.stack([p2x, p2y, p2z], axis=-1)

    w1pt = jnp.transpose(W1[:, :_COORD])
    w1xt = jnp.transpose(W1[:, _COORD:])
    table, knn = _run_table_knn(
        p1, x1, w1pt, w1xt, px, py, pz,
        p2x[:, :, None], p2y[:, :, None], p2z[:, :, None])
    idx_flat = knn.reshape(_P)

    G = _sc_gather(table, idx_flat)

    p2f = p2.reshape(_B * _M, _COORD)
    new_features = _run_mlp(G, p2f, w1pt,
                            gamma1[None, :], beta1[None, :],
                            jnp.transpose(W2),
                            gamma2[None, :], beta2[None, :])
    return p2, new_features
